# baseline (speedup 1.0000x reference)
import functools

import jax
import jax.numpy as jnp
import numpy as np
from jax import lax
from jax.experimental import pallas as pl
from jax.experimental.pallas import tpu as pltpu
from jax.experimental.pallas import tpu_sc as plsc

B, T, N, C = 2, 8, 1000, 8
H = 64
D = 2 * H
E = 32000
S = 4
M1 = 32
W1R = S * M1
WMR = S * H
NW = 32
GW = E // NW
KC = 40
NCH = GW // KC
EC = 4000
RT = 250
RC = 2000


def _silu(x):
    return x * jax.nn.sigmoid(x)


@functools.cache
def _sc_gather_kernel():
    mesh = plsc.VectorSubcoreMesh(core_axis_name="c", subcore_axis_name="s")

    @functools.partial(
        pl.kernel,
        out_type=[jax.ShapeDtypeStruct((E, W1R), jnp.bfloat16),
                  jax.ShapeDtypeStruct((E, W1R), jnp.bfloat16)],
        mesh=mesh,
        scratch_types=[pltpu.VMEM((NCH, KC), jnp.int32),
                       pltpu.VMEM((NCH, KC), jnp.int32)]
        + [pltpu.VMEM((KC, W1R), jnp.bfloat16)] * 6
        + [pltpu.SemaphoreType.DMA] * 6,
        compiler_params=pltpu.CompilerParams(use_tc_tiling_on_sc=False),
    )
    def _sc_gather(tabA, tabB, idxA, idxB, gA, gB, iva, ivb,
                   ba0, ba1, ba2, bb0, bb1, bb2,
                   sg0, sg1, sg2, ss0, ss1, ss2):
        wid = lax.axis_index("c") * 16 + lax.axis_index("s")
        base = wid * GW
        BA, BB = [ba0, ba1, ba2], [bb0, bb1, bb2]
        SG, SS = [sg0, sg1, sg2], [ss0, ss1, ss2]
        pltpu.sync_copy(idxA.at[wid], iva)
        pltpu.sync_copy(idxB.at[wid], ivb)

        def gath(j, s):
            pltpu.async_copy(tabA.at[iva.at[j]], BA[s], SG[s])
            pltpu.async_copy(tabB.at[ivb.at[j]], BB[s], SG[s])

        def drain(buf, sem):
            pltpu.make_async_copy(tabA.at[pl.ds(0, KC)], buf, sem).wait()

        for s in range(3):
            gath(s, s)

        def body(jj, _):
            for s in range(3):
                j = jj * 3 + s

                @pl.when(j < NCH)
                def _(j=j, s=s):
                    @pl.when(j >= 3)
                    def _():
                        drain(BA[s], SS[s])
                        drain(BB[s], SS[s])
                    drain(BA[s], SG[s])
                    drain(BB[s], SG[s])
                    pltpu.async_copy(BA[s], gA.at[pl.ds(base + j * KC, KC)],
                                     SS[s])
                    pltpu.async_copy(BB[s], gB.at[pl.ds(base + j * KC, KC)],
                                     SS[s])

                    @pl.when(j + 3 < NCH)
                    def _():
                        gath(j + 3, s)
            return 0

        lax.fori_loop(0, (NCH + 2) // 3, body, 0)
        for s in range(3):
            drain(BA[s], SS[s])
            drain(BB[s], SS[s])

    return _sc_gather


@functools.cache
def _sc_scatter_kernel():
    mesh = plsc.VectorSubcoreMesh(core_axis_name="c", subcore_axis_name="s")

    @functools.partial(
        pl.kernel,
        out_type=jax.ShapeDtypeStruct((2, 2, N, W1R), jnp.float32),
        mesh=mesh,
        scratch_types=[pltpu.VMEM((NCH, KC), jnp.int32),
                       pltpu.VMEM((NCH, KC), jnp.int32)]
        + [pltpu.VMEM((KC, W1R), jnp.float32)] * 6
        + [pltpu.SemaphoreType.DMA] * 6
        + [pltpu.VMEM_SHARED((2 * N, W1R), jnp.float32)],
        compiler_params=pltpu.CompilerParams(use_tc_tiling_on_sc=False),
    )
    def _sc_scatter(gm0, gm1, idxD, idxD2, zz, out, iv, iv2,
                    b00, b01, b02, b10, b11, b12,
                    sf0, sf1, sf2, sw0, sw1, sw2, acc):
        cid = lax.axis_index("c")
        sid = lax.axis_index("s")
        wid = cid * 16 + sid
        base = wid * GW
        B0 = [b00, b01, b02]
        B1 = [b10, b11, b12]
        SF, SW = [sf0, sf1, sf2], [sw0, sw1, sw2]
        stripe = 200
        @pl.when(sid < 5)
        def _():
            pltpu.sync_copy(zz.at[pl.ds(sid * stripe, stripe)],
                            acc.at[pl.ds(sid * stripe, stripe)])
        @pl.when(jnp.logical_and(sid >= 5, sid < 10))
        def _():
            pltpu.sync_copy(zz.at[pl.ds((sid - 5) * stripe, stripe)],
                            acc.at[pl.ds(N + (sid - 5) * stripe, stripe)])
        pltpu.sync_copy(idxD.at[wid], iv)
        pltpu.sync_copy(idxD2.at[wid], iv2)

        def fetch(j, s):
            pltpu.async_copy(gm0.at[pl.ds(base + j * KC, KC)], B0[s], SF[s])
            pltpu.async_copy(gm1.at[pl.ds(base + j * KC, KC)], B1[s], SF[s])

        def drain(buf, sem):
            pltpu.make_async_copy(gm0.at[pl.ds(0, KC)], buf, sem).wait()

        for s in range(3):
            fetch(s, s)
        plsc.subcore_barrier()

        def body(jj, _):
            for s in range(3):
                j = jj * 3 + s

                @pl.when(j < NCH)
                def _(j=j, s=s):
                    @pl.when(j >= 3)
                    def _():
                        drain(B0[s], SW[s])
                        drain(B1[s], SW[s])
                    drain(B0[s], SF[s])
                    drain(B1[s], SF[s])
                    pltpu.async_copy(B0[s], acc.at[iv.at[j]], SW[s], add=True)
                    pltpu.async_copy(B1[s], acc.at[iv2.at[j]], SW[s], add=True)

                    @pl.when(j + 3 < NCH)
                    def _():
                        fetch(j + 3, s)
            return 0

        lax.fori_loop(0, (NCH + 2) // 3, body, 0)
        for s in range(3):
            drain(B0[s], SW[s])
            drain(B1[s], SW[s])
        plsc.subcore_barrier()
        @pl.when(sid < 5)
        def _():
            pltpu.sync_copy(acc.at[pl.ds(sid * stripe, stripe)],
                            out.at[cid, 0, pl.ds(sid * stripe, stripe)])
        @pl.when(jnp.logical_and(sid >= 5, sid < 10))
        def _():
            pltpu.sync_copy(acc.at[pl.ds(N + (sid - 5) * stripe, stripe)],
                            out.at[cid, 1, pl.ds((sid - 5) * stripe, stripe)])

    return _sc_scatter


def _gather_impl(tA, tB, idxA, idxB):
    return _sc_gather_kernel()(tA, tB, idxA, idxB)


def _scatter_impl(gm0, gm1, idxD, idxD2, zz):
    return _sc_scatter_kernel()(gm0, gm1, idxD, idxD2, zz)


def _full(a):
    r = len(a.shape)
    return pl.BlockSpec(a.shape, lambda *g: (0,) * r)


def _enc_body(x_ref, emb, encW, encb, inWf, inbf, Pf, Pbf, inWb, inbb, Pb_, Pbb,
              of, ob):
    x = x_ref[0]
    h = jnp.dot(x, encW[...], preferred_element_type=jnp.float32) + encb[...] \
        + emb[...]
    xtf = jnp.dot(h, inWf[...], preferred_element_type=jnp.float32) + inbf[...]
    of[0] = jnp.dot(xtf, Pf[...], preferred_element_type=jnp.float32) + Pbf[...]
    xtb = jnp.dot(h, inWb[...], preferred_element_type=jnp.float32) + inbb[...]
    ob[0] = jnp.dot(xtb, Pb_[...], preferred_element_type=jnp.float32) + Pbb[...]


def _edge_body(gA, gB, W2big, b2big, Gmat, gb4, Rm, out0, out1):
    m = _silu(gA[...].astype(jnp.float32) + gB[...].astype(jnp.float32))
    m2 = _silu(jnp.dot(m, W2big[...], preferred_element_type=jnp.float32)
               + b2big[...])
    sig = jax.nn.sigmoid(jnp.dot(m2, Gmat[...],
                                 preferred_element_type=jnp.float32) + gb4[...])
    res = m2 * jnp.dot(sig, Rm[...], preferred_element_type=jnp.float32)
    out0[...] = res[:, :W1R]
    out1[...] = res[:, W1R:]


def _upd_body(agg2, st, pU, pS, pAn, pBn, Ua, Us, U2, u2b, Ss, W1a, W1b,
              ns_out, tA_out, tB_out):
    agg = jnp.concatenate([agg2[0, 0] + agg2[1, 0], agg2[0, 1] + agg2[1, 1]],
                          -1)
    state = st[...]
    nss, tas, tbs = [], [], []
    for s in range(S):
        d = s // 2
        a = agg[:, H * s:H * (s + 1)]
        s0 = state[:, H * s:H * (s + 1)]
        u = _silu(jnp.dot(a, Ua[d], preferred_element_type=jnp.float32)
                  + jnp.dot(s0, Us[d], preferred_element_type=jnp.float32)
                  + pU[:, H * s:H * (s + 1)])
        o = jnp.dot(u, U2[d], preferred_element_type=jnp.float32) + u2b[d] \
            + jnp.dot(s0, Ss[d], preferred_element_type=jnp.float32) \
            + pS[:, H * s:H * (s + 1)]
        ns = s0 + o
        nss.append(ns)
        tas.append(jnp.dot(ns, W1a[d], preferred_element_type=jnp.float32)
                   + pAn[:, M1 * s:M1 * (s + 1)])
        tbs.append(jnp.dot(ns, W1b[d], preferred_element_type=jnp.float32)
                   + pBn[:, M1 * s:M1 * (s + 1)])
    ns_out[...] = jnp.concatenate(nss, -1)
    tA_out[...] = jnp.concatenate(tas, -1).astype(jnp.bfloat16)
    tB_out[...] = jnp.concatenate(tbs, -1).astype(jnp.bfloat16)


def _temporal_body(stin, x0r, skW, skb, tlnw, tlnb, taiW, taib, taoW, taob,
                   gW, gb, out):
    sti = stin[...].reshape(RT * T, D)
    x0 = x0r[...].reshape(RT * T, C)
    sk = jnp.dot(x0, skW[...], preferred_element_type=jnp.float32) + skb[...]
    st = sti + sk
    mean = jnp.mean(st, -1, keepdims=True)
    std = jnp.sqrt(jnp.mean((st - mean) ** 2, -1, keepdims=True))
    std = jnp.clip(std, 1e-8, 1e19)
    stn = (st - mean) / (std + 1e-4) * tlnw[...] + tlnb[...]
    qkv = jnp.dot(stn, taiW[...], preferred_element_type=jnp.float32) + taib[...]
    q, k, v = qkv[:, :D], qkv[:, D:2 * D], qkv[:, 2 * D:]
    outs = []
    hd = D // 4
    for h in range(4):
        qh = q[:, h * hd:(h + 1) * hd].reshape(RT, T, hd)
        kh = k[:, h * hd:(h + 1) * hd].reshape(RT, T, hd)
        vh = v[:, h * hd:(h + 1) * hd].reshape(RT, T, hd)
        sc = lax.dot_general(qh, kh, (((2,), (2,)), ((0,), (0,))),
                             preferred_element_type=jnp.float32) \
            * (1.0 / np.sqrt(hd))
        sc = jax.nn.softmax(sc, -1)
        oh = lax.dot_general(sc, vh, (((2,), (1,)), ((0,), (0,))),
                             preferred_element_type=jnp.float32)
        outs.append(oh.reshape(RT * T, hd))
    attn = jnp.concatenate(outs, -1)
    attn = jnp.dot(attn, taoW[...], preferred_element_type=jnp.float32) + taob[...]
    st2 = stn + attn
    gate = jax.nn.sigmoid(jnp.dot(st2, gW[...],
                                  preferred_element_type=jnp.float32) + gb[...])
    out[...] = (gate * st2 + (1.0 - gate) * sk).reshape(RT, T, D)


def _station_body(s2in, slnw, slnb, saiW, saib, saoW, saob, out):
    xx = s2in[0]
    mean = jnp.mean(xx, -1, keepdims=True)
    var = jnp.mean((xx - mean) ** 2, -1, keepdims=True)
    xn = (xx - mean) * lax.rsqrt(var + 1e-5) * slnw[...] + slnb[...]
    qkv = jnp.dot(xn, saiW[...], preferred_element_type=jnp.float32) + saib[...]
    q, k, v = qkv[:, :D], qkv[:, D:2 * D], qkv[:, 2 * D:]
    res = []
    hd = D // 2
    for h in range(2):
        qh = q[:, h * hd:(h + 1) * hd].astype(jnp.bfloat16)
        kh = k[:, h * hd:(h + 1) * hd].astype(jnp.bfloat16)
        vh = v[:, h * hd:(h + 1) * hd].astype(jnp.bfloat16)
        sc = lax.dot_general(qh, kh, (((1,), (1,)), ((), ())),
                             preferred_element_type=jnp.float32) \
            * (1.0 / np.sqrt(hd))
        sc = jax.nn.softmax(sc, -1)
        res.append(jnp.dot(sc.astype(jnp.bfloat16), vh,
                           preferred_element_type=jnp.float32))
    o = jnp.concatenate(res, -1)
    out[0] = jnp.dot(o, saoW[...], preferred_element_type=jnp.float32) + saob[...]


def _ro1_body(a, b, W1, b1, r_out, sum_out, sumsq_out):
    xx = a[...] + b[...]
    r = jnp.dot(xx, W1[...], preferred_element_type=jnp.float32) + b1[...]
    r_out[...] = r
    sum_out[...] = jnp.sum(r, 0, keepdims=True).reshape(1, 1, H)
    sumsq_out[...] = jnp.sum(r * r, 0, keepdims=True).reshape(1, 1, H)


def _ro2_body(r_in, sums, sumsqs, bnw, bnb, W2, b2r, locw, locb, sclw, sclb,
              out):
    cnt = float(B * T * N)
    nb = sums.shape[0]
    mean = jnp.sum(sums[...].reshape(nb, H), 0, keepdims=True) / cnt
    var = jnp.sum(sumsqs[...].reshape(nb, H), 0, keepdims=True) / cnt \
        - mean * mean
    rn = (r_in[...] - mean) * lax.rsqrt(var + 1e-5) * bnw[...] + bnb[...]
    rn = _silu(rn)
    r2 = jnp.dot(rn, W2[...], preferred_element_type=jnp.float32) + b2r[...]
    loc = jnp.sum(r2 * locw[...], -1, keepdims=True) + locb[...]
    sc = jnp.sum(r2 * sclw[...], -1, keepdims=True) + sclb[...]
    sp = jnp.maximum(sc, 0.0) + jnp.log1p(jnp.exp(-jnp.abs(sc)))
    out[...] = jnp.concatenate([loc, sp], -1)


def kernel(x, edge_index, params):
    p = params
    src, dst = edge_index[0], edge_index[1]
    f32 = jnp.float32

    def row2(a):
        return a.reshape(1, -1)

    pf, pb = p['fwd'], p['bwd']

    def projmat(dp):
        return jnp.concatenate([dp['msg_W1'][H:2 * H],
                                dp['msg_W1'][3 * H:4 * H],
                                dp['upd_W1'][2 * H:3 * H],
                                dp['skip_W'][H:2 * H]], axis=1)

    def projbias(dp):
        return jnp.concatenate([dp['msg_b1'], jnp.zeros((M1,), f32),
                                dp['upd_b1'], dp['skip_b']]).reshape(1, -1)

    def stk(fn):
        return jnp.stack([fn(pf), fn(pb)], 0)

    Uast = stk(lambda dp: dp['upd_W1'][0:H])
    Usst = stk(lambda dp: dp['upd_W1'][H:2 * H])
    U2st = stk(lambda dp: dp['upd_W2'])
    u2bst = stk(lambda dp: row2(dp['upd_b2']))
    Ssst = stk(lambda dp: dp['skip_W'][0:H])
    W1ast = stk(lambda dp: dp['msg_W1'][0:H])
    W1bst = stk(lambda dp: dp['msg_W1'][2 * H:3 * H])

    W2big = jnp.zeros((W1R, WMR), f32)
    b2big = jnp.zeros((1, WMR), f32)
    Gmat = jnp.zeros((WMR, S), f32)
    gb4 = jnp.zeros((1, S), f32)
    Rm = jnp.zeros((S, WMR), f32)
    for s in range(S):
        dp = pf if s < 2 else pb
        W2big = W2big.at[M1 * s:M1 * (s + 1), H * s:H * (s + 1)].set(dp['msg_W2'])
        b2big = b2big.at[0, H * s:H * (s + 1)].set(dp['msg_b2'])
        Gmat = Gmat.at[H * s:H * (s + 1), s].set(dp['gate_W'][:, 0])
        gb4 = gb4.at[0, s].set(dp['gate_b'][0])
        Rm = Rm.at[s, H * s:H * (s + 1)].set(1.0)

    xr = x.reshape(B * T, N, C)
    enc_call = pl.pallas_call(
        _enc_body,
        grid=(B * T,),
        in_specs=[pl.BlockSpec((1, N, C), lambda g: (g, 0, 0)),
                  _full(p['node_emb']), _full(p['enc_W']),
                  pl.BlockSpec((1, H), lambda g: (0, 0)),
                  _full(pf['in_W']), pl.BlockSpec((1, H), lambda g: (0, 0)),
                  pl.BlockSpec((H, 192), lambda g: (0, 0)),
                  pl.BlockSpec((1, 192), lambda g: (0, 0)),
                  _full(pb['in_W']), pl.BlockSpec((1, H), lambda g: (0, 0)),
                  pl.BlockSpec((H, 192), lambda g: (0, 0)),
                  pl.BlockSpec((1, 192), lambda g: (0, 0))],
        out_specs=[pl.BlockSpec((1, N, 192), lambda g: (g, 0, 0)),
                   pl.BlockSpec((1, N, 192), lambda g: (g, 0, 0))],
        out_shape=[jax.ShapeDtypeStruct((B * T, N, 192), f32),
                   jax.ShapeDtypeStruct((B * T, N, 192), f32)],
    )
    prf, prb = enc_call(xr, p['node_emb'], p['enc_W'], row2(p['enc_b']),
                        pf['in_W'], row2(pf['in_b']), projmat(pf), projbias(pf),
                        pb['in_W'], row2(pb['in_b']), projmat(pb), projbias(pb))
    prf = prf.reshape(B, T, N, 192)
    prb = prb.reshape(B, T, N, 192)

    def steps(sl):
        df, db = prf[..., sl], jnp.flip(prb[..., sl], 1)
        return jnp.concatenate([df[0], df[1], db[0], db[1]], axis=-1)

    stepA = steps(np.s_[:M1])
    stepB_ = steps(np.s_[M1:2 * M1])
    stepU = steps(np.s_[64:128])
    stepS = steps(np.s_[128:192])
    stepAn = jnp.concatenate([stepA[1:], stepA[-1:]], 0)
    stepBn = jnp.concatenate([stepB_[1:], stepB_[-1:]], 0)

    idxA = dst.reshape(NW, NCH, KC)
    idxB = src.reshape(NW, NCH, KC)
    idxD2 = idxA + N
    zz = jnp.zeros((N, W1R), f32)

    edge_call = pl.pallas_call(
        _edge_body,
        grid=(E // EC,),
        in_specs=[pl.BlockSpec((EC, W1R), lambda g: (g, 0)),
                  pl.BlockSpec((EC, W1R), lambda g: (g, 0)),
                  _full(W2big), _full(b2big), _full(Gmat), _full(gb4),
                  _full(Rm)],
        out_specs=[pl.BlockSpec((EC, W1R), lambda g: (g, 0)),
                   pl.BlockSpec((EC, W1R), lambda g: (g, 0))],
        out_shape=[jax.ShapeDtypeStruct((E, W1R), f32),
                   jax.ShapeDtypeStruct((E, W1R), f32)],
    )

    upd_call = pl.pallas_call(
        _upd_body,
        in_specs=[_full(jax.ShapeDtypeStruct((2, 2, N, W1R), f32)),
                  _full(jax.ShapeDtypeStruct((N, WMR), f32)),
                  _full(jax.ShapeDtypeStruct((N, WMR), f32)),
                  _full(jax.ShapeDtypeStruct((N, WMR), f32)),
                  _full(jax.ShapeDtypeStruct((N, W1R), f32)),
                  _full(jax.ShapeDtypeStruct((N, W1R), f32)),
                  _full(Uast), _full(Usst), _full(U2st),
                  _full(u2bst.reshape(2, 1, H)), _full(Ssst),
                  _full(W1ast), _full(W1bst)],
        out_specs=[pl.BlockSpec((N, WMR), lambda: (0, 0)),
                   pl.BlockSpec((N, W1R), lambda: (0, 0)),
                   pl.BlockSpec((N, W1R), lambda: (0, 0))],
        out_shape=[jax.ShapeDtypeStruct((N, WMR), f32),
                   jax.ShapeDtypeStruct((N, W1R), jnp.bfloat16),
                   jax.ShapeDtypeStruct((N, W1R), jnp.bfloat16)],
    )
    u2b3 = u2bst.reshape(2, 1, H)

    def body(carry, xs):
        state, tA, tB = carry
        pU_k, pS_k, pAn, pBn = xs
        gA, gB = _gather_impl(tA, tB, idxA, idxB)
        gm0, gm1 = edge_call(gA, gB, W2big, b2big, Gmat, gb4, Rm)
        agg2 = _scatter_impl(gm0, gm1, idxA, idxD2, zz)
        ns, tA2, tB2 = upd_call(agg2, state, pU_k, pS_k, pAn, pBn,
                                Uast, Usst, U2st, u2b3, Ssst, W1ast, W1bst)
        return (ns, tA2, tB2), ns

    state0 = jnp.zeros((N, WMR), f32)
    _, states = lax.scan(
        body,
        (state0, stepA[0].astype(jnp.bfloat16), stepB_[0].astype(jnp.bfloat16)),
        (stepU, stepS, stepAn, stepBn))

    sfT = states[..., :D]
    sbT = jnp.flip(states[..., D:], 0)
    stall = jnp.stack([
        jnp.concatenate([sfT[..., 0:H], sbT[..., 0:H]], -1),
        jnp.concatenate([sfT[..., H:D], sbT[..., H:D]], -1)], 0)

    stin = stall.transpose(0, 2, 1, 3).reshape(B * N, T, D)
    x0r = x.transpose(0, 2, 1, 3).reshape(B * N, T, C)
    temporal_call = pl.pallas_call(
        _temporal_body,
        grid=(B * N // RT,),
        in_specs=[pl.BlockSpec((RT, T, D), lambda g: (g, 0, 0)),
                  pl.BlockSpec((RT, T, C), lambda g: (g, 0, 0)),
                  _full(p['skip_W']), pl.BlockSpec((1, D), lambda g: (0, 0)),
                  pl.BlockSpec((1, D), lambda g: (0, 0)),
                  pl.BlockSpec((1, D), lambda g: (0, 0)),
                  _full(p['ta_in_W']),
                  pl.BlockSpec((1, 3 * D), lambda g: (0, 0)),
                  _full(p['ta_out_W']),
                  pl.BlockSpec((1, D), lambda g: (0, 0)),
                  _full(p['gate_W']),
                  pl.BlockSpec((1, D), lambda g: (0, 0))],
        out_specs=pl.BlockSpec((RT, T, D), lambda g: (g, 0, 0)),
        out_shape=jax.ShapeDtypeStruct((B * N, T, D), f32),
    )
    st = temporal_call(stin, x0r, p['skip_W'], row2(p['skip_b']),
                       row2(p['tln_w']), row2(p['tln_b']),
                       p['ta_in_W'], row2(p['ta_in_b']),
                       p['ta_out_W'], row2(p['ta_out_b']),
                       p['gate_W'], row2(p['gate_b']))

    s2 = st.reshape(B, N, T, D).transpose(0, 2, 1, 3).reshape(B * T, N, D)
    station_call = pl.pallas_call(
        _station_body,
        grid=(B * T,),
        in_specs=[pl.BlockSpec((1, N, D), lambda g: (g, 0, 0)),
                  pl.BlockSpec((1, D), lambda g: (0, 0)),
                  pl.BlockSpec((1, D), lambda g: (0, 0)),
                  _full(p['sa_in_W']),
                  pl.BlockSpec((1, 3 * D), lambda g: (0, 0)),
                  _full(p['sa_out_W']),
                  pl.BlockSpec((1, D), lambda g: (0, 0))],
        out_specs=pl.BlockSpec((1, N, D), lambda g: (g, 0, 0)),
        out_shape=jax.ShapeDtypeStruct((B * T, N, D), f32),
    )
    stat = station_call(s2, row2(p['sln_w']), row2(p['sln_b']),
                        p['sa_in_W'], row2(p['sa_in_b']),
                        p['sa_out_W'], row2(p['sa_out_b']))

    st_btnd = st.reshape(B, N, T, D).transpose(0, 2, 1, 3).reshape(B * T * N, D)
    statf = stat.reshape(B * T * N, D)
    NB = B * T * N // RC
    ro1_call = pl.pallas_call(
        _ro1_body,
        grid=(NB,),
        in_specs=[pl.BlockSpec((RC, D), lambda g: (g, 0)),
                  pl.BlockSpec((RC, D), lambda g: (g, 0)),
                  _full(p['ro_W1']), pl.BlockSpec((1, H), lambda g: (0, 0))],
        out_specs=[pl.BlockSpec((RC, H), lambda g: (g, 0)),
                   pl.BlockSpec((1, 1, H), lambda g: (g, 0, 0)),
                   pl.BlockSpec((1, 1, H), lambda g: (g, 0, 0))],
        out_shape=[jax.ShapeDtypeStruct((B * T * N, H), f32),
                   jax.ShapeDtypeStruct((NB, 1, H), f32),
                   jax.ShapeDtypeStruct((NB, 1, H), f32)],
    )
    r, sums, sumsqs = ro1_call(st_btnd, statf, p['ro_W1'], row2(p['ro_b1']))

    ro2_call = pl.pallas_call(
        _ro2_body,
        grid=(NB,),
        in_specs=[pl.BlockSpec((RC, H), lambda g: (g, 0)),
                  _full(sums), _full(sumsqs),
                  pl.BlockSpec((1, H), lambda g: (0, 0)),
                  pl.BlockSpec((1, H), lambda g: (0, 0)),
                  _full(p['ro_W2']),
                  pl.BlockSpec((1, H), lambda g: (0, 0)),
                  pl.BlockSpec((1, H), lambda g: (0, 0)),
                  pl.BlockSpec((1, 1), lambda g: (0, 0)),
                  pl.BlockSpec((1, H), lambda g: (0, 0)),
                  pl.BlockSpec((1, 1), lambda g: (0, 0))],
        out_specs=pl.BlockSpec((RC, 2), lambda g: (g, 0)),
        out_shape=jax.ShapeDtypeStruct((B * T * N, 2), f32),
    )
    outf = ro2_call(r, sums, sumsqs, row2(p['bn_w']), row2(p['bn_b']),
                    p['ro_W2'], row2(p['ro_b2']),
                    p['loc_W'].reshape(1, H), p['loc_b'].reshape(1, 1),
                    p['scale_W'].reshape(1, H), p['scale_b'].reshape(1, 1))
    return outf.reshape(B, T, N, 2)


# speedup vs baseline: 1.4805x; 1.4805x over previous
import functools

import jax
import jax.numpy as jnp
import numpy as np
from jax import lax
from jax.experimental import pallas as pl
from jax.experimental.pallas import tpu as pltpu
from jax.experimental.pallas import tpu_sc as plsc

B, T, N, C = 2, 8, 1000, 8
H = 64
D = 2 * H
E = 32000
S = 4
M1 = 32
W1R = S * M1
WMR = S * H
NW = 32
GW = E // NW
KC = 40
NCH = GW // KC
EC = 4000
RT = 250
RC = 2000


def _silu(x):
    return x * jax.nn.sigmoid(x)


@functools.cache
def _sc_gather_kernel():
    mesh = plsc.VectorSubcoreMesh(core_axis_name="c", subcore_axis_name="s")

    @functools.partial(
        pl.kernel,
        out_type=[jax.ShapeDtypeStruct((E, W1R), jnp.float32),
                  jax.ShapeDtypeStruct((E, W1R), jnp.float32)],
        mesh=mesh,
        scratch_types=[pltpu.VMEM((NCH, KC), jnp.int32),
                       pltpu.VMEM((NCH, KC), jnp.int32)]
        + [pltpu.VMEM((KC, W1R), jnp.float32)] * 6
        + [pltpu.SemaphoreType.DMA] * 6,
        compiler_params=pltpu.CompilerParams(use_tc_tiling_on_sc=False),
    )
    def _sc_gather(tabA, tabB, idxA, idxB, gA, gB, iva, ivb,
                   ba0, ba1, ba2, bb0, bb1, bb2,
                   sg0, sg1, sg2, ss0, ss1, ss2):
        wid = lax.axis_index("c") * 16 + lax.axis_index("s")
        base = wid * GW
        BA, BB = [ba0, ba1, ba2], [bb0, bb1, bb2]
        SG, SS = [sg0, sg1, sg2], [ss0, ss1, ss2]
        pltpu.sync_copy(idxA.at[wid], iva)
        pltpu.sync_copy(idxB.at[wid], ivb)

        def gath(j, s):
            pltpu.async_copy(tabA.at[iva.at[j]], BA[s], SG[s])
            pltpu.async_copy(tabB.at[ivb.at[j]], BB[s], SG[s])

        def drain(buf, sem):
            pltpu.make_async_copy(tabA.at[pl.ds(0, KC)], buf, sem).wait()

        for s in range(3):
            gath(s, s)

        def body(jj, _):
            for s in range(3):
                j = jj * 3 + s

                @pl.when(j < NCH)
                def _(j=j, s=s):
                    @pl.when(j >= 3)
                    def _():
                        drain(BA[s], SS[s])
                        drain(BB[s], SS[s])
                    drain(BA[s], SG[s])
                    drain(BB[s], SG[s])
                    pltpu.async_copy(BA[s], gA.at[pl.ds(base + j * KC, KC)],
                                     SS[s])
                    pltpu.async_copy(BB[s], gB.at[pl.ds(base + j * KC, KC)],
                                     SS[s])

                    @pl.when(j + 3 < NCH)
                    def _():
                        gath(j + 3, s)
            return 0

        lax.fori_loop(0, (NCH + 2) // 3, body, 0)
        for s in range(3):
            drain(BA[s], SS[s])
            drain(BB[s], SS[s])

    return _sc_gather


@functools.cache
def _sc_scatter_kernel():
    mesh = plsc.VectorSubcoreMesh(core_axis_name="c", subcore_axis_name="s")

    @functools.partial(
        pl.kernel,
        out_type=jax.ShapeDtypeStruct((2, 2, N, W1R), jnp.float32),
        mesh=mesh,
        scratch_types=[pltpu.VMEM((NCH, KC), jnp.int32),
                       pltpu.VMEM((NCH, KC), jnp.int32)]
        + [pltpu.VMEM((KC, W1R), jnp.float32)] * 6
        + [pltpu.SemaphoreType.DMA] * 6
        + [pltpu.VMEM_SHARED((2 * N, W1R), jnp.float32)],
        compiler_params=pltpu.CompilerParams(use_tc_tiling_on_sc=False),
    )
    def _sc_scatter(gm0, gm1, idxD, idxD2, zz, out, iv, iv2,
                    b00, b01, b02, b10, b11, b12,
                    sf0, sf1, sf2, sw0, sw1, sw2, acc):
        cid = lax.axis_index("c")
        sid = lax.axis_index("s")
        wid = cid * 16 + sid
        base = wid * GW
        B0 = [b00, b01, b02]
        B1 = [b10, b11, b12]
        SF, SW = [sf0, sf1, sf2], [sw0, sw1, sw2]
        stripe = 200
        @pl.when(sid < 5)
        def _():
            pltpu.sync_copy(zz.at[pl.ds(sid * stripe, stripe)],
                            acc.at[pl.ds(sid * stripe, stripe)])
        @pl.when(jnp.logical_and(sid >= 5, sid < 10))
        def _():
            pltpu.sync_copy(zz.at[pl.ds((sid - 5) * stripe, stripe)],
                            acc.at[pl.ds(N + (sid - 5) * stripe, stripe)])
        pltpu.sync_copy(idxD.at[wid], iv)
        pltpu.sync_copy(idxD2.at[wid], iv2)

        def fetch(j, s):
            pltpu.async_copy(gm0.at[pl.ds(base + j * KC, KC)], B0[s], SF[s])
            pltpu.async_copy(gm1.at[pl.ds(base + j * KC, KC)], B1[s], SF[s])

        def drain(buf, sem):
            pltpu.make_async_copy(gm0.at[pl.ds(0, KC)], buf, sem).wait()

        for s in range(3):
            fetch(s, s)
        plsc.subcore_barrier()

        def body(jj, _):
            for s in range(3):
                j = jj * 3 + s

                @pl.when(j < NCH)
                def _(j=j, s=s):
                    @pl.when(j >= 3)
                    def _():
                        drain(B0[s], SW[s])
                        drain(B1[s], SW[s])
                    drain(B0[s], SF[s])
                    drain(B1[s], SF[s])
                    pltpu.async_copy(B0[s], acc.at[iv.at[j]], SW[s], add=True)
                    pltpu.async_copy(B1[s], acc.at[iv2.at[j]], SW[s], add=True)

                    @pl.when(j + 3 < NCH)
                    def _():
                        fetch(j + 3, s)
            return 0

        lax.fori_loop(0, (NCH + 2) // 3, body, 0)
        for s in range(3):
            drain(B0[s], SW[s])
            drain(B1[s], SW[s])
        plsc.subcore_barrier()
        @pl.when(sid < 5)
        def _():
            pltpu.sync_copy(acc.at[pl.ds(sid * stripe, stripe)],
                            out.at[cid, 0, pl.ds(sid * stripe, stripe)])
        @pl.when(jnp.logical_and(sid >= 5, sid < 10))
        def _():
            pltpu.sync_copy(acc.at[pl.ds(N + (sid - 5) * stripe, stripe)],
                            out.at[cid, 1, pl.ds((sid - 5) * stripe, stripe)])

    return _sc_scatter


def _gather_impl(tA, tB, idxA, idxB):
    return _sc_gather_kernel()(tA, tB, idxA, idxB)


def _scatter_impl(gm0, gm1, idxD, idxD2, zz):
    return _sc_scatter_kernel()(gm0, gm1, idxD, idxD2, zz)


def _full(a):
    r = len(a.shape)
    return pl.BlockSpec(a.shape, lambda *g: (0,) * r)


def _enc_body(x_ref, emb, encW, encb, inWf, inbf, Pf, Pbf, inWb, inbb, Pb_, Pbb,
              of, ob):
    x = x_ref[0]
    h = jnp.dot(x, encW[...], preferred_element_type=jnp.float32) + encb[...] \
        + emb[...]
    xtf = jnp.dot(h, inWf[...], preferred_element_type=jnp.float32) + inbf[...]
    of[0] = jnp.dot(xtf, Pf[...], preferred_element_type=jnp.float32) + Pbf[...]
    xtb = jnp.dot(h, inWb[...], preferred_element_type=jnp.float32) + inbb[...]
    ob[0] = jnp.dot(xtb, Pb_[...], preferred_element_type=jnp.float32) + Pbb[...]


def _edge_body(gA, gB, W2big, b2big, Gmat, gb4, Rm, out0, out1):
    m = _silu(gA[...] + gB[...])
    m2 = _silu(jnp.dot(m, W2big[...], preferred_element_type=jnp.float32)
               + b2big[...])
    sig = jax.nn.sigmoid(jnp.dot(m2, Gmat[...],
                                 preferred_element_type=jnp.float32) + gb4[...])
    res = m2 * jnp.dot(sig, Rm[...], preferred_element_type=jnp.float32)
    out0[...] = res[:, :W1R]
    out1[...] = res[:, W1R:]


def _upd_body(agg2, st, pU, pS, pAn, pBn, Ua, Us, U2, u2b, Ss, W1a, W1b,
              ns_out, tA_out, tB_out):
    agg = jnp.concatenate([agg2[0, 0] + agg2[1, 0], agg2[0, 1] + agg2[1, 1]],
                          -1)
    state = st[...]
    nss, tas, tbs = [], [], []
    for s in range(S):
        d = s // 2
        a = agg[:, H * s:H * (s + 1)]
        s0 = state[:, H * s:H * (s + 1)]
        u = _silu(jnp.dot(a, Ua[d], preferred_element_type=jnp.float32)
                  + jnp.dot(s0, Us[d], preferred_element_type=jnp.float32)
                  + pU[:, H * s:H * (s + 1)])
        o = jnp.dot(u, U2[d], preferred_element_type=jnp.float32) + u2b[d] \
            + jnp.dot(s0, Ss[d], preferred_element_type=jnp.float32) \
            + pS[:, H * s:H * (s + 1)]
        ns = s0 + o
        nss.append(ns)
        tas.append(jnp.dot(ns, W1a[d], preferred_element_type=jnp.float32)
                   + pAn[:, M1 * s:M1 * (s + 1)])
        tbs.append(jnp.dot(ns, W1b[d], preferred_element_type=jnp.float32)
                   + pBn[:, M1 * s:M1 * (s + 1)])
    ns_out[...] = jnp.concatenate(nss, -1)
    tA_out[...] = jnp.concatenate(tas, -1)
    tB_out[...] = jnp.concatenate(tbs, -1)


def _temporal_body(stin, x0r, skW, skb, tlnw, tlnb, taiW, taib, taoW, taob,
                   gW, gb, out):
    sti = stin[...].reshape(RT * T, D)
    x0 = x0r[...].reshape(RT * T, C)
    sk = jnp.dot(x0, skW[...], preferred_element_type=jnp.float32) + skb[...]
    st = sti + sk
    mean = jnp.mean(st, -1, keepdims=True)
    std = jnp.sqrt(jnp.mean((st - mean) ** 2, -1, keepdims=True))
    std = jnp.clip(std, 1e-8, 1e19)
    stn = (st - mean) / (std + 1e-4) * tlnw[...] + tlnb[...]
    qkv = jnp.dot(stn, taiW[...], preferred_element_type=jnp.float32) + taib[...]
    q, k, v = qkv[:, :D], qkv[:, D:2 * D], qkv[:, 2 * D:]
    outs = []
    hd = D // 4
    for h in range(4):
        qh = q[:, h * hd:(h + 1) * hd].reshape(RT, T, hd)
        kh = k[:, h * hd:(h + 1) * hd].reshape(RT, T, hd)
        vh = v[:, h * hd:(h + 1) * hd].reshape(RT, T, hd)
        sc = lax.dot_general(qh, kh, (((2,), (2,)), ((0,), (0,))),
                             preferred_element_type=jnp.float32) \
            * (1.0 / np.sqrt(hd))
        sc = jax.nn.softmax(sc, -1)
        oh = lax.dot_general(sc, vh, (((2,), (1,)), ((0,), (0,))),
                             preferred_element_type=jnp.float32)
        outs.append(oh.reshape(RT * T, hd))
    attn = jnp.concatenate(outs, -1)
    attn = jnp.dot(attn, taoW[...], preferred_element_type=jnp.float32) + taob[...]
    st2 = stn + attn
    gate = jax.nn.sigmoid(jnp.dot(st2, gW[...],
                                  preferred_element_type=jnp.float32) + gb[...])
    out[...] = (gate * st2 + (1.0 - gate) * sk).reshape(RT, T, D)


def _station_body(s2in, slnw, slnb, saiW, saib, saoW, saob, out):
    xx = s2in[0]
    mean = jnp.mean(xx, -1, keepdims=True)
    var = jnp.mean((xx - mean) ** 2, -1, keepdims=True)
    xn = (xx - mean) * lax.rsqrt(var + 1e-5) * slnw[...] + slnb[...]
    qkv = jnp.dot(xn, saiW[...], preferred_element_type=jnp.float32) + saib[...]
    q, k, v = qkv[:, :D], qkv[:, D:2 * D], qkv[:, 2 * D:]
    res = []
    hd = D // 2
    for h in range(2):
        qh = q[:, h * hd:(h + 1) * hd].astype(jnp.bfloat16)
        kh = k[:, h * hd:(h + 1) * hd].astype(jnp.bfloat16)
        vh = v[:, h * hd:(h + 1) * hd].astype(jnp.bfloat16)
        sc = lax.dot_general(qh, kh, (((1,), (1,)), ((), ())),
                             preferred_element_type=jnp.float32) \
            * (1.0 / np.sqrt(hd))
        sc = jax.nn.softmax(sc, -1)
        res.append(jnp.dot(sc.astype(jnp.bfloat16), vh,
                           preferred_element_type=jnp.float32))
    o = jnp.concatenate(res, -1)
    out[0] = jnp.dot(o, saoW[...], preferred_element_type=jnp.float32) + saob[...]


def _ro1_body(a, b, W1, b1, r_out, sum_out, sumsq_out):
    xx = a[...] + b[...]
    r = jnp.dot(xx, W1[...], preferred_element_type=jnp.float32) + b1[...]
    r_out[...] = r
    sum_out[...] = jnp.sum(r, 0, keepdims=True).reshape(1, 1, H)
    sumsq_out[...] = jnp.sum(r * r, 0, keepdims=True).reshape(1, 1, H)


def _ro2_body(r_in, sums, sumsqs, bnw, bnb, W2, b2r, locw, locb, sclw, sclb,
              out):
    cnt = float(B * T * N)
    nb = sums.shape[0]
    mean = jnp.sum(sums[...].reshape(nb, H), 0, keepdims=True) / cnt
    var = jnp.sum(sumsqs[...].reshape(nb, H), 0, keepdims=True) / cnt \
        - mean * mean
    rn = (r_in[...] - mean) * lax.rsqrt(var + 1e-5) * bnw[...] + bnb[...]
    rn = _silu(rn)
    r2 = jnp.dot(rn, W2[...], preferred_element_type=jnp.float32) + b2r[...]
    loc = jnp.sum(r2 * locw[...], -1, keepdims=True) + locb[...]
    sc = jnp.sum(r2 * sclw[...], -1, keepdims=True) + sclb[...]
    sp = jnp.maximum(sc, 0.0) + jnp.log1p(jnp.exp(-jnp.abs(sc)))
    out[...] = jnp.concatenate([loc, sp], -1)


def kernel(x, edge_index, params):
    p = params
    src, dst = edge_index[0], edge_index[1]
    f32 = jnp.float32

    def row2(a):
        return a.reshape(1, -1)

    pf, pb = p['fwd'], p['bwd']

    def projmat(dp):
        return jnp.concatenate([dp['msg_W1'][H:2 * H],
                                dp['msg_W1'][3 * H:4 * H],
                                dp['upd_W1'][2 * H:3 * H],
                                dp['skip_W'][H:2 * H]], axis=1)

    def projbias(dp):
        return jnp.concatenate([dp['msg_b1'], jnp.zeros((M1,), f32),
                                dp['upd_b1'], dp['skip_b']]).reshape(1, -1)

    def stk(fn):
        return jnp.stack([fn(pf), fn(pb)], 0)

    Uast = stk(lambda dp: dp['upd_W1'][0:H])
    Usst = stk(lambda dp: dp['upd_W1'][H:2 * H])
    U2st = stk(lambda dp: dp['upd_W2'])
    u2bst = stk(lambda dp: row2(dp['upd_b2']))
    Ssst = stk(lambda dp: dp['skip_W'][0:H])
    W1ast = stk(lambda dp: dp['msg_W1'][0:H])
    W1bst = stk(lambda dp: dp['msg_W1'][2 * H:3 * H])

    W2big = jnp.zeros((W1R, WMR), f32)
    b2big = jnp.zeros((1, WMR), f32)
    Gmat = jnp.zeros((WMR, S), f32)
    gb4 = jnp.zeros((1, S), f32)
    Rm = jnp.zeros((S, WMR), f32)
    for s in range(S):
        dp = pf if s < 2 else pb
        W2big = W2big.at[M1 * s:M1 * (s + 1), H * s:H * (s + 1)].set(dp['msg_W2'])
        b2big = b2big.at[0, H * s:H * (s + 1)].set(dp['msg_b2'])
        Gmat = Gmat.at[H * s:H * (s + 1), s].set(dp['gate_W'][:, 0])
        gb4 = gb4.at[0, s].set(dp['gate_b'][0])
        Rm = Rm.at[s, H * s:H * (s + 1)].set(1.0)

    xr = x.reshape(B * T, N, C)
    enc_call = pl.pallas_call(
        _enc_body,
        grid=(B * T,),
        in_specs=[pl.BlockSpec((1, N, C), lambda g: (g, 0, 0)),
                  _full(p['node_emb']), _full(p['enc_W']),
                  pl.BlockSpec((1, H), lambda g: (0, 0)),
                  _full(pf['in_W']), pl.BlockSpec((1, H), lambda g: (0, 0)),
                  pl.BlockSpec((H, 192), lambda g: (0, 0)),
                  pl.BlockSpec((1, 192), lambda g: (0, 0)),
                  _full(pb['in_W']), pl.BlockSpec((1, H), lambda g: (0, 0)),
                  pl.BlockSpec((H, 192), lambda g: (0, 0)),
                  pl.BlockSpec((1, 192), lambda g: (0, 0))],
        out_specs=[pl.BlockSpec((1, N, 192), lambda g: (g, 0, 0)),
                   pl.BlockSpec((1, N, 192), lambda g: (g, 0, 0))],
        out_shape=[jax.ShapeDtypeStruct((B * T, N, 192), f32),
                   jax.ShapeDtypeStruct((B * T, N, 192), f32)],
    )
    prf, prb = enc_call(xr, p['node_emb'], p['enc_W'], row2(p['enc_b']),
                        pf['in_W'], row2(pf['in_b']), projmat(pf), projbias(pf),
                        pb['in_W'], row2(pb['in_b']), projmat(pb), projbias(pb))
    prf = prf.reshape(B, T, N, 192)
    prb = prb.reshape(B, T, N, 192)

    def steps(sl):
        df, db = prf[..., sl], jnp.flip(prb[..., sl], 1)
        return jnp.concatenate([df[0], df[1], db[0], db[1]], axis=-1)

    stepA = steps(np.s_[:M1])
    stepB_ = steps(np.s_[M1:2 * M1])
    stepU = steps(np.s_[64:128])
    stepS = steps(np.s_[128:192])
    stepAn = jnp.concatenate([stepA[1:], stepA[-1:]], 0)
    stepBn = jnp.concatenate([stepB_[1:], stepB_[-1:]], 0)

    idxA = dst.reshape(NW, NCH, KC)
    idxB = src.reshape(NW, NCH, KC)
    idxD2 = idxA + N
    zz = jnp.zeros((N, W1R), f32)

    edge_call = pl.pallas_call(
        _edge_body,
        grid=(E // EC,),
        in_specs=[pl.BlockSpec((EC, W1R), lambda g: (g, 0)),
                  pl.BlockSpec((EC, W1R), lambda g: (g, 0)),
                  _full(W2big), _full(b2big), _full(Gmat), _full(gb4),
                  _full(Rm)],
        out_specs=[pl.BlockSpec((EC, W1R), lambda g: (g, 0)),
                   pl.BlockSpec((EC, W1R), lambda g: (g, 0))],
        out_shape=[jax.ShapeDtypeStruct((E, W1R), f32),
                   jax.ShapeDtypeStruct((E, W1R), f32)],
    )

    upd_call = pl.pallas_call(
        _upd_body,
        in_specs=[_full(jax.ShapeDtypeStruct((2, 2, N, W1R), f32)),
                  _full(jax.ShapeDtypeStruct((N, WMR), f32)),
                  _full(jax.ShapeDtypeStruct((N, WMR), f32)),
                  _full(jax.ShapeDtypeStruct((N, WMR), f32)),
                  _full(jax.ShapeDtypeStruct((N, W1R), f32)),
                  _full(jax.ShapeDtypeStruct((N, W1R), f32)),
                  _full(Uast), _full(Usst), _full(U2st),
                  _full(u2bst.reshape(2, 1, H)), _full(Ssst),
                  _full(W1ast), _full(W1bst)],
        out_specs=[pl.BlockSpec((N, WMR), lambda: (0, 0)),
                   pl.BlockSpec((N, W1R), lambda: (0, 0)),
                   pl.BlockSpec((N, W1R), lambda: (0, 0))],
        out_shape=[jax.ShapeDtypeStruct((N, WMR), f32),
                   jax.ShapeDtypeStruct((N, W1R), f32),
                   jax.ShapeDtypeStruct((N, W1R), f32)],
    )
    u2b3 = u2bst.reshape(2, 1, H)

    def body(carry, xs):
        state, tA, tB = carry
        pU_k, pS_k, pAn, pBn = xs
        gA, gB = _gather_impl(tA, tB, idxA, idxB)
        gm0, gm1 = edge_call(gA, gB, W2big, b2big, Gmat, gb4, Rm)
        agg2 = _scatter_impl(gm0, gm1, idxA, idxD2, zz)
        ns, tA2, tB2 = upd_call(agg2, state, pU_k, pS_k, pAn, pBn,
                                Uast, Usst, U2st, u2b3, Ssst, W1ast, W1bst)
        return (ns, tA2, tB2), ns

    state0 = jnp.zeros((N, WMR), f32)
    _, states = lax.scan(body, (state0, stepA[0], stepB_[0]),
                         (stepU, stepS, stepAn, stepBn))

    sfT = states[..., :D]
    sbT = jnp.flip(states[..., D:], 0)
    stall = jnp.stack([
        jnp.concatenate([sfT[..., 0:H], sbT[..., 0:H]], -1),
        jnp.concatenate([sfT[..., H:D], sbT[..., H:D]], -1)], 0)

    stin = stall.transpose(0, 2, 1, 3).reshape(B * N, T, D)
    x0r = x.transpose(0, 2, 1, 3).reshape(B * N, T, C)
    temporal_call = pl.pallas_call(
        _temporal_body,
        grid=(B * N // RT,),
        in_specs=[pl.BlockSpec((RT, T, D), lambda g: (g, 0, 0)),
                  pl.BlockSpec((RT, T, C), lambda g: (g, 0, 0)),
                  _full(p['skip_W']), pl.BlockSpec((1, D), lambda g: (0, 0)),
                  pl.BlockSpec((1, D), lambda g: (0, 0)),
                  pl.BlockSpec((1, D), lambda g: (0, 0)),
                  _full(p['ta_in_W']),
                  pl.BlockSpec((1, 3 * D), lambda g: (0, 0)),
                  _full(p['ta_out_W']),
                  pl.BlockSpec((1, D), lambda g: (0, 0)),
                  _full(p['gate_W']),
                  pl.BlockSpec((1, D), lambda g: (0, 0))],
        out_specs=pl.BlockSpec((RT, T, D), lambda g: (g, 0, 0)),
        out_shape=jax.ShapeDtypeStruct((B * N, T, D), f32),
    )
    st = temporal_call(stin, x0r, p['skip_W'], row2(p['skip_b']),
                       row2(p['tln_w']), row2(p['tln_b']),
                       p['ta_in_W'], row2(p['ta_in_b']),
                       p['ta_out_W'], row2(p['ta_out_b']),
                       p['gate_W'], row2(p['gate_b']))

    s2 = st.reshape(B, N, T, D).transpose(0, 2, 1, 3).reshape(B * T, N, D)
    station_call = pl.pallas_call(
        _station_body,
        grid=(B * T,),
        in_specs=[pl.BlockSpec((1, N, D), lambda g: (g, 0, 0)),
                  pl.BlockSpec((1, D), lambda g: (0, 0)),
                  pl.BlockSpec((1, D), lambda g: (0, 0)),
                  _full(p['sa_in_W']),
                  pl.BlockSpec((1, 3 * D), lambda g: (0, 0)),
                  _full(p['sa_out_W']),
                  pl.BlockSpec((1, D), lambda g: (0, 0))],
        out_specs=pl.BlockSpec((1, N, D), lambda g: (g, 0, 0)),
        out_shape=jax.ShapeDtypeStruct((B * T, N, D), f32),
    )
    stat = station_call(s2, row2(p['sln_w']), row2(p['sln_b']),
                        p['sa_in_W'], row2(p['sa_in_b']),
                        p['sa_out_W'], row2(p['sa_out_b']))

    st_btnd = st.reshape(B, N, T, D).transpose(0, 2, 1, 3).reshape(B * T * N, D)
    statf = stat.reshape(B * T * N, D)
    NB = B * T * N // RC
    ro1_call = pl.pallas_call(
        _ro1_body,
        grid=(NB,),
        in_specs=[pl.BlockSpec((RC, D), lambda g: (g, 0)),
                  pl.BlockSpec((RC, D), lambda g: (g, 0)),
                  _full(p['ro_W1']), pl.BlockSpec((1, H), lambda g: (0, 0))],
        out_specs=[pl.BlockSpec((RC, H), lambda g: (g, 0)),
                   pl.BlockSpec((1, 1, H), lambda g: (g, 0, 0)),
                   pl.BlockSpec((1, 1, H), lambda g: (g, 0, 0))],
        out_shape=[jax.ShapeDtypeStruct((B * T * N, H), f32),
                   jax.ShapeDtypeStruct((NB, 1, H), f32),
                   jax.ShapeDtypeStruct((NB, 1, H), f32)],
    )
    r, sums, sumsqs = ro1_call(st_btnd, statf, p['ro_W1'], row2(p['ro_b1']))

    ro2_call = pl.pallas_call(
        _ro2_body,
        grid=(NB,),
        in_specs=[pl.BlockSpec((RC, H), lambda g: (g, 0)),
                  _full(sums), _full(sumsqs),
                  pl.BlockSpec((1, H), lambda g: (0, 0)),
                  pl.BlockSpec((1, H), lambda g: (0, 0)),
                  _full(p['ro_W2']),
                  pl.BlockSpec((1, H), lambda g: (0, 0)),
                  pl.BlockSpec((1, H), lambda g: (0, 0)),
                  pl.BlockSpec((1, 1), lambda g: (0, 0)),
                  pl.BlockSpec((1, H), lambda g: (0, 0)),
                  pl.BlockSpec((1, 1), lambda g: (0, 0))],
        out_specs=pl.BlockSpec((RC, 2), lambda g: (g, 0)),
        out_shape=jax.ShapeDtypeStruct((B * T * N, 2), f32),
    )
    outf = ro2_call(r, sums, sumsqs, row2(p['bn_w']), row2(p['bn_b']),
                    p['ro_W2'], row2(p['ro_b2']),
                    p['loc_W'].reshape(1, H), p['loc_b'].reshape(1, 1),
                    p['scale_W'].reshape(1, H), p['scale_b'].reshape(1, 1))
    return outf.reshape(B, T, N, 2)


# speedup vs baseline: 1.4812x; 1.0005x over previous
import functools

import jax
import jax.numpy as jnp
import numpy as np
from jax import lax
from jax.experimental import pallas as pl
from jax.experimental.pallas import tpu as pltpu
from jax.experimental.pallas import tpu_sc as plsc

B, T, N, C = 2, 8, 1000, 8
H = 64
D = 2 * H
E = 32000
S = 4
M1 = 32
W1R = S * M1
WMR = S * H
NW = 32
GW = E // NW
KC = 40
NCH = GW // KC
EC = 4000
RT = 250
RC = 2000


def _silu(x):
    return x * jax.nn.sigmoid(x)


@functools.cache
def _sc_gather_kernel():
    mesh = plsc.VectorSubcoreMesh(core_axis_name="c", subcore_axis_name="s")

    @functools.partial(
        pl.kernel,
        out_type=[jax.ShapeDtypeStruct((E, W1R), jnp.float32),
                  jax.ShapeDtypeStruct((E, W1R), jnp.float32)],
        mesh=mesh,
        scratch_types=[pltpu.VMEM((NCH, KC), jnp.int32),
                       pltpu.VMEM((NCH, KC), jnp.int32)]
        + [pltpu.VMEM((KC, W1R), jnp.float32)] * 6
        + [pltpu.SemaphoreType.DMA] * 6,
        compiler_params=pltpu.CompilerParams(use_tc_tiling_on_sc=False),
    )
    def _sc_gather(tabA, tabB, idxA, idxB, gA, gB, iva, ivb,
                   ba0, ba1, ba2, bb0, bb1, bb2,
                   sg0, sg1, sg2, ss0, ss1, ss2):
        wid = lax.axis_index("c") * 16 + lax.axis_index("s")
        base = wid * GW
        BA, BB = [ba0, ba1, ba2], [bb0, bb1, bb2]
        SG, SS = [sg0, sg1, sg2], [ss0, ss1, ss2]
        pltpu.sync_copy(idxA.at[wid], iva)
        pltpu.sync_copy(idxB.at[wid], ivb)

        def gath(j, s):
            pltpu.async_copy(tabA.at[iva.at[j]], BA[s], SG[s])
            pltpu.async_copy(tabB.at[ivb.at[j]], BB[s], SG[s])

        def drain(buf, sem):
            pltpu.make_async_copy(tabA.at[pl.ds(0, KC)], buf, sem).wait()

        for s in range(3):
            gath(s, s)

        def body(jj, _):
            for s in range(3):
                j = jj * 3 + s

                @pl.when(j < NCH)
                def _(j=j, s=s):
                    @pl.when(j >= 3)
                    def _():
                        drain(BA[s], SS[s])
                        drain(BB[s], SS[s])
                    drain(BA[s], SG[s])
                    drain(BB[s], SG[s])
                    pltpu.async_copy(BA[s], gA.at[pl.ds(base + j * KC, KC)],
                                     SS[s])
                    pltpu.async_copy(BB[s], gB.at[pl.ds(base + j * KC, KC)],
                                     SS[s])

                    @pl.when(j + 3 < NCH)
                    def _():
                        gath(j + 3, s)
            return 0

        lax.fori_loop(0, (NCH + 2) // 3, body, 0)
        for s in range(3):
            drain(BA[s], SS[s])
            drain(BB[s], SS[s])

    return _sc_gather


@functools.cache
def _sc_scatter_kernel():
    mesh = plsc.VectorSubcoreMesh(core_axis_name="c", subcore_axis_name="s")

    @functools.partial(
        pl.kernel,
        out_type=jax.ShapeDtypeStruct((2, 2, N, W1R), jnp.float32),
        mesh=mesh,
        scratch_types=[pltpu.VMEM((NCH, KC), jnp.int32),
                       pltpu.VMEM((NCH, KC), jnp.int32)]
        + [pltpu.VMEM((KC, W1R), jnp.float32)] * 6
        + [pltpu.SemaphoreType.DMA] * 6
        + [pltpu.VMEM_SHARED((2 * N, W1R), jnp.float32)],
        compiler_params=pltpu.CompilerParams(use_tc_tiling_on_sc=False),
    )
    def _sc_scatter(gm0, gm1, idxD, idxD2, zz, out, iv, iv2,
                    b00, b01, b02, b10, b11, b12,
                    sf0, sf1, sf2, sw0, sw1, sw2, acc):
        cid = lax.axis_index("c")
        sid = lax.axis_index("s")
        wid = cid * 16 + sid
        base = wid * GW
        B0 = [b00, b01, b02]
        B1 = [b10, b11, b12]
        SF, SW = [sf0, sf1, sf2], [sw0, sw1, sw2]
        stripe = 200
        @pl.when(sid < 5)
        def _():
            pltpu.sync_copy(zz.at[pl.ds(sid * stripe, stripe)],
                            acc.at[pl.ds(sid * stripe, stripe)])
        @pl.when(jnp.logical_and(sid >= 5, sid < 10))
        def _():
            pltpu.sync_copy(zz.at[pl.ds((sid - 5) * stripe, stripe)],
                            acc.at[pl.ds(N + (sid - 5) * stripe, stripe)])
        pltpu.sync_copy(idxD.at[wid], iv)
        pltpu.sync_copy(idxD2.at[wid], iv2)

        def fetch(j, s):
            pltpu.async_copy(gm0.at[pl.ds(base + j * KC, KC)], B0[s], SF[s])
            pltpu.async_copy(gm1.at[pl.ds(base + j * KC, KC)], B1[s], SF[s])

        def drain(buf, sem):
            pltpu.make_async_copy(gm0.at[pl.ds(0, KC)], buf, sem).wait()

        for s in range(3):
            fetch(s, s)
        plsc.subcore_barrier()

        def body(jj, _):
            for s in range(3):
                j = jj * 3 + s

                @pl.when(j < NCH)
                def _(j=j, s=s):
                    @pl.when(j >= 3)
                    def _():
                        drain(B0[s], SW[s])
                        drain(B1[s], SW[s])
                    drain(B0[s], SF[s])
                    drain(B1[s], SF[s])
                    pltpu.async_copy(B0[s], acc.at[iv.at[j]], SW[s], add=True)
                    pltpu.async_copy(B1[s], acc.at[iv2.at[j]], SW[s], add=True)

                    @pl.when(j + 3 < NCH)
                    def _():
                        fetch(j + 3, s)
            return 0

        lax.fori_loop(0, (NCH + 2) // 3, body, 0)
        for s in range(3):
            drain(B0[s], SW[s])
            drain(B1[s], SW[s])
        plsc.subcore_barrier()
        @pl.when(sid < 5)
        def _():
            pltpu.sync_copy(acc.at[pl.ds(sid * stripe, stripe)],
                            out.at[cid, 0, pl.ds(sid * stripe, stripe)])
        @pl.when(jnp.logical_and(sid >= 5, sid < 10))
        def _():
            pltpu.sync_copy(acc.at[pl.ds(N + (sid - 5) * stripe, stripe)],
                            out.at[cid, 1, pl.ds((sid - 5) * stripe, stripe)])

    return _sc_scatter


def _gather_impl(tA, tB, idxA, idxB):
    return _sc_gather_kernel()(tA, tB, idxA, idxB)


def _scatter_impl(gm0, gm1, idxD, idxD2, zz):
    return _sc_scatter_kernel()(gm0, gm1, idxD, idxD2, zz)


def _full(a):
    r = len(a.shape)
    return pl.BlockSpec(a.shape, lambda *g: (0,) * r)


def _enc_body(x_ref, emb, encW, encb, inWf, inbf, Pf, Pbf, inWb, inbb, Pb_, Pbb,
              of, ob):
    x = x_ref[0]
    h = jnp.dot(x, encW[...], preferred_element_type=jnp.float32) + encb[...] \
        + emb[...]
    xtf = jnp.dot(h, inWf[...], preferred_element_type=jnp.float32) + inbf[...]
    of[0] = jnp.dot(xtf, Pf[...], preferred_element_type=jnp.float32) + Pbf[...]
    xtb = jnp.dot(h, inWb[...], preferred_element_type=jnp.float32) + inbb[...]
    ob[0] = jnp.dot(xtb, Pb_[...], preferred_element_type=jnp.float32) + Pbb[...]


def _edge_body(gA, gB, W2big, b2big, Gmat, gb4, Rm, out0, out1):
    m = _silu(gA[...] + gB[...])
    m2 = _silu(jnp.dot(m.astype(jnp.bfloat16), W2big[...],
                       preferred_element_type=jnp.float32) + b2big[...])
    sig = jax.nn.sigmoid(jnp.dot(m2.astype(jnp.bfloat16), Gmat[...],
                                 preferred_element_type=jnp.float32) + gb4[...])
    res = m2 * jnp.dot(sig, Rm[...], preferred_element_type=jnp.float32)
    out0[...] = res[:, :W1R]
    out1[...] = res[:, W1R:]


def _upd_body(agg2, st, pU, pS, pAn, pBn, Ua, Us, U2, u2b, Ss, W1a, W1b,
              ns_out, tA_out, tB_out):
    agg = jnp.concatenate([agg2[0, 0] + agg2[1, 0], agg2[0, 1] + agg2[1, 1]],
                          -1)
    state = st[...]
    nss, tas, tbs = [], [], []
    for s in range(S):
        d = s // 2
        a = agg[:, H * s:H * (s + 1)]
        s0 = state[:, H * s:H * (s + 1)]
        u = _silu(jnp.dot(a, Ua[d], preferred_element_type=jnp.float32)
                  + jnp.dot(s0, Us[d], preferred_element_type=jnp.float32)
                  + pU[:, H * s:H * (s + 1)])
        o = jnp.dot(u, U2[d], preferred_element_type=jnp.float32) + u2b[d] \
            + jnp.dot(s0, Ss[d], preferred_element_type=jnp.float32) \
            + pS[:, H * s:H * (s + 1)]
        ns = s0 + o
        nss.append(ns)
        tas.append(jnp.dot(ns, W1a[d], preferred_element_type=jnp.float32)
                   + pAn[:, M1 * s:M1 * (s + 1)])
        tbs.append(jnp.dot(ns, W1b[d], preferred_element_type=jnp.float32)
                   + pBn[:, M1 * s:M1 * (s + 1)])
    ns_out[...] = jnp.concatenate(nss, -1)
    tA_out[...] = jnp.concatenate(tas, -1)
    tB_out[...] = jnp.concatenate(tbs, -1)


def _temporal_body(stin, x0r, skW, skb, tlnw, tlnb, taiW, taib, taoW, taob,
                   gW, gb, out):
    sti = stin[...].reshape(RT * T, D)
    x0 = x0r[...].reshape(RT * T, C)
    sk = jnp.dot(x0, skW[...], preferred_element_type=jnp.float32) + skb[...]
    st = sti + sk
    mean = jnp.mean(st, -1, keepdims=True)
    std = jnp.sqrt(jnp.mean((st - mean) ** 2, -1, keepdims=True))
    std = jnp.clip(std, 1e-8, 1e19)
    stn = (st - mean) / (std + 1e-4) * tlnw[...] + tlnb[...]
    qkv = jnp.dot(stn, taiW[...], preferred_element_type=jnp.float32) + taib[...]
    q, k, v = qkv[:, :D], qkv[:, D:2 * D], qkv[:, 2 * D:]
    outs = []
    hd = D // 4
    for h in range(4):
        qh = q[:, h * hd:(h + 1) * hd].reshape(RT, T, hd)
        kh = k[:, h * hd:(h + 1) * hd].reshape(RT, T, hd)
        vh = v[:, h * hd:(h + 1) * hd].reshape(RT, T, hd)
        sc = lax.dot_general(qh, kh, (((2,), (2,)), ((0,), (0,))),
                             preferred_element_type=jnp.float32) \
            * (1.0 / np.sqrt(hd))
        sc = jax.nn.softmax(sc, -1)
        oh = lax.dot_general(sc, vh, (((2,), (1,)), ((0,), (0,))),
                             preferred_element_type=jnp.float32)
        outs.append(oh.reshape(RT * T, hd))
    attn = jnp.concatenate(outs, -1)
    attn = jnp.dot(attn, taoW[...], preferred_element_type=jnp.float32) + taob[...]
    st2 = stn + attn
    gate = jax.nn.sigmoid(jnp.dot(st2, gW[...],
                                  preferred_element_type=jnp.float32) + gb[...])
    out[...] = (gate * st2 + (1.0 - gate) * sk).reshape(RT, T, D)


def _station_body(s2in, slnw, slnb, saiW, saib, saoW, saob, out):
    xx = s2in[0]
    mean = jnp.mean(xx, -1, keepdims=True)
    var = jnp.mean((xx - mean) ** 2, -1, keepdims=True)
    xn = (xx - mean) * lax.rsqrt(var + 1e-5) * slnw[...] + slnb[...]
    qkv = jnp.dot(xn, saiW[...], preferred_element_type=jnp.float32) + saib[...]
    q, k, v = qkv[:, :D], qkv[:, D:2 * D], qkv[:, 2 * D:]
    res = []
    hd = D // 2
    for h in range(2):
        qh = q[:, h * hd:(h + 1) * hd].astype(jnp.bfloat16)
        kh = k[:, h * hd:(h + 1) * hd].astype(jnp.bfloat16)
        vh = v[:, h * hd:(h + 1) * hd].astype(jnp.bfloat16)
        sc = lax.dot_general(qh, kh, (((1,), (1,)), ((), ())),
                             preferred_element_type=jnp.float32) \
            * (1.0 / np.sqrt(hd))
        sc = jax.nn.softmax(sc, -1)
        res.append(jnp.dot(sc.astype(jnp.bfloat16), vh,
                           preferred_element_type=jnp.float32))
    o = jnp.concatenate(res, -1)
    out[0] = jnp.dot(o, saoW[...], preferred_element_type=jnp.float32) + saob[...]


def _ro1_body(a, b, W1, b1, r_out, sum_out, sumsq_out):
    xx = a[...] + b[...]
    r = jnp.dot(xx, W1[...], preferred_element_type=jnp.float32) + b1[...]
    r_out[...] = r
    sum_out[...] = jnp.sum(r, 0, keepdims=True).reshape(1, 1, H)
    sumsq_out[...] = jnp.sum(r * r, 0, keepdims=True).reshape(1, 1, H)


def _ro2_body(r_in, sums, sumsqs, bnw, bnb, W2, b2r, locw, locb, sclw, sclb,
              out):
    cnt = float(B * T * N)
    nb = sums.shape[0]
    mean = jnp.sum(sums[...].reshape(nb, H), 0, keepdims=True) / cnt
    var = jnp.sum(sumsqs[...].reshape(nb, H), 0, keepdims=True) / cnt \
        - mean * mean
    rn = (r_in[...] - mean) * lax.rsqrt(var + 1e-5) * bnw[...] + bnb[...]
    rn = _silu(rn)
    r2 = jnp.dot(rn, W2[...], preferred_element_type=jnp.float32) + b2r[...]
    loc = jnp.sum(r2 * locw[...], -1, keepdims=True) + locb[...]
    sc = jnp.sum(r2 * sclw[...], -1, keepdims=True) + sclb[...]
    sp = jnp.maximum(sc, 0.0) + jnp.log1p(jnp.exp(-jnp.abs(sc)))
    out[...] = jnp.concatenate([loc, sp], -1)


def kernel(x, edge_index, params):
    p = params
    src, dst = edge_index[0], edge_index[1]
    f32 = jnp.float32

    def row2(a):
        return a.reshape(1, -1)

    pf, pb = p['fwd'], p['bwd']

    def projmat(dp):
        return jnp.concatenate([dp['msg_W1'][H:2 * H],
                                dp['msg_W1'][3 * H:4 * H],
                                dp['upd_W1'][2 * H:3 * H],
                                dp['skip_W'][H:2 * H]], axis=1)

    def projbias(dp):
        return jnp.concatenate([dp['msg_b1'], jnp.zeros((M1,), f32),
                                dp['upd_b1'], dp['skip_b']]).reshape(1, -1)

    def stk(fn):
        return jnp.stack([fn(pf), fn(pb)], 0)

    Uast = stk(lambda dp: dp['upd_W1'][0:H])
    Usst = stk(lambda dp: dp['upd_W1'][H:2 * H])
    U2st = stk(lambda dp: dp['upd_W2'])
    u2bst = stk(lambda dp: row2(dp['upd_b2']))
    Ssst = stk(lambda dp: dp['skip_W'][0:H])
    W1ast = stk(lambda dp: dp['msg_W1'][0:H])
    W1bst = stk(lambda dp: dp['msg_W1'][2 * H:3 * H])

    W2big = jnp.zeros((W1R, WMR), f32)
    b2big = jnp.zeros((1, WMR), f32)
    Gmat = jnp.zeros((WMR, S), f32)
    gb4 = jnp.zeros((1, S), f32)
    Rm = jnp.zeros((S, WMR), f32)
    for s in range(S):
        dp = pf if s < 2 else pb
        W2big = W2big.at[M1 * s:M1 * (s + 1), H * s:H * (s + 1)].set(dp['msg_W2'])
        b2big = b2big.at[0, H * s:H * (s + 1)].set(dp['msg_b2'])
        Gmat = Gmat.at[H * s:H * (s + 1), s].set(dp['gate_W'][:, 0])
        gb4 = gb4.at[0, s].set(dp['gate_b'][0])
        Rm = Rm.at[s, H * s:H * (s + 1)].set(1.0)
    W2big = W2big.astype(jnp.bfloat16)
    Gmat = Gmat.astype(jnp.bfloat16)

    xr = x.reshape(B * T, N, C)
    enc_call = pl.pallas_call(
        _enc_body,
        grid=(B * T,),
        in_specs=[pl.BlockSpec((1, N, C), lambda g: (g, 0, 0)),
                  _full(p['node_emb']), _full(p['enc_W']),
                  pl.BlockSpec((1, H), lambda g: (0, 0)),
                  _full(pf['in_W']), pl.BlockSpec((1, H), lambda g: (0, 0)),
                  pl.BlockSpec((H, 192), lambda g: (0, 0)),
                  pl.BlockSpec((1, 192), lambda g: (0, 0)),
                  _full(pb['in_W']), pl.BlockSpec((1, H), lambda g: (0, 0)),
                  pl.BlockSpec((H, 192), lambda g: (0, 0)),
                  pl.BlockSpec((1, 192), lambda g: (0, 0))],
        out_specs=[pl.BlockSpec((1, N, 192), lambda g: (g, 0, 0)),
                   pl.BlockSpec((1, N, 192), lambda g: (g, 0, 0))],
        out_shape=[jax.ShapeDtypeStruct((B * T, N, 192), f32),
                   jax.ShapeDtypeStruct((B * T, N, 192), f32)],
    )
    prf, prb = enc_call(xr, p['node_emb'], p['enc_W'], row2(p['enc_b']),
                        pf['in_W'], row2(pf['in_b']), projmat(pf), projbias(pf),
                        pb['in_W'], row2(pb['in_b']), projmat(pb), projbias(pb))
    prf = prf.reshape(B, T, N, 192)
    prb = prb.reshape(B, T, N, 192)

    def steps(sl):
        df, db = prf[..., sl], jnp.flip(prb[..., sl], 1)
        return jnp.concatenate([df[0], df[1], db[0], db[1]], axis=-1)

    stepA = steps(np.s_[:M1])
    stepB_ = steps(np.s_[M1:2 * M1])
    stepU = steps(np.s_[64:128])
    stepS = steps(np.s_[128:192])
    stepAn = jnp.concatenate([stepA[1:], stepA[-1:]], 0)
    stepBn = jnp.concatenate([stepB_[1:], stepB_[-1:]], 0)

    idxA = dst.reshape(NW, NCH, KC)
    idxB = src.reshape(NW, NCH, KC)
    idxD2 = idxA + N
    zz = jnp.zeros((N, W1R), f32)

    edge_call = pl.pallas_call(
        _edge_body,
        grid=(E // EC,),
        in_specs=[pl.BlockSpec((EC, W1R), lambda g: (g, 0)),
                  pl.BlockSpec((EC, W1R), lambda g: (g, 0)),
                  _full(W2big), _full(b2big), _full(Gmat), _full(gb4),
                  _full(Rm)],
        out_specs=[pl.BlockSpec((EC, W1R), lambda g: (g, 0)),
                   pl.BlockSpec((EC, W1R), lambda g: (g, 0))],
        out_shape=[jax.ShapeDtypeStruct((E, W1R), f32),
                   jax.ShapeDtypeStruct((E, W1R), f32)],
    )

    upd_call = pl.pallas_call(
        _upd_body,
        in_specs=[_full(jax.ShapeDtypeStruct((2, 2, N, W1R), f32)),
                  _full(jax.ShapeDtypeStruct((N, WMR), f32)),
                  _full(jax.ShapeDtypeStruct((N, WMR), f32)),
                  _full(jax.ShapeDtypeStruct((N, WMR), f32)),
                  _full(jax.ShapeDtypeStruct((N, W1R), f32)),
                  _full(jax.ShapeDtypeStruct((N, W1R), f32)),
                  _full(Uast), _full(Usst), _full(U2st),
                  _full(u2bst.reshape(2, 1, H)), _full(Ssst),
                  _full(W1ast), _full(W1bst)],
        out_specs=[pl.BlockSpec((N, WMR), lambda: (0, 0)),
                   pl.BlockSpec((N, W1R), lambda: (0, 0)),
                   pl.BlockSpec((N, W1R), lambda: (0, 0))],
        out_shape=[jax.ShapeDtypeStruct((N, WMR), f32),
                   jax.ShapeDtypeStruct((N, W1R), f32),
                   jax.ShapeDtypeStruct((N, W1R), f32)],
    )
    u2b3 = u2bst.reshape(2, 1, H)

    def body(carry, xs):
        state, tA, tB = carry
        pU_k, pS_k, pAn, pBn = xs
        gA, gB = _gather_impl(tA, tB, idxA, idxB)
        gm0, gm1 = edge_call(gA, gB, W2big, b2big, Gmat, gb4, Rm)
        agg2 = _scatter_impl(gm0, gm1, idxA, idxD2, zz)
        ns, tA2, tB2 = upd_call(agg2, state, pU_k, pS_k, pAn, pBn,
                                Uast, Usst, U2st, u2b3, Ssst, W1ast, W1bst)
        return (ns, tA2, tB2), ns

    state0 = jnp.zeros((N, WMR), f32)
    _, states = lax.scan(body, (state0, stepA[0], stepB_[0]),
                         (stepU, stepS, stepAn, stepBn))

    sfT = states[..., :D]
    sbT = jnp.flip(states[..., D:], 0)
    stall = jnp.stack([
        jnp.concatenate([sfT[..., 0:H], sbT[..., 0:H]], -1),
        jnp.concatenate([sfT[..., H:D], sbT[..., H:D]], -1)], 0)

    stin = stall.transpose(0, 2, 1, 3).reshape(B * N, T, D)
    x0r = x.transpose(0, 2, 1, 3).reshape(B * N, T, C)
    temporal_call = pl.pallas_call(
        _temporal_body,
        grid=(B * N // RT,),
        in_specs=[pl.BlockSpec((RT, T, D), lambda g: (g, 0, 0)),
                  pl.BlockSpec((RT, T, C), lambda g: (g, 0, 0)),
                  _full(p['skip_W']), pl.BlockSpec((1, D), lambda g: (0, 0)),
                  pl.BlockSpec((1, D), lambda g: (0, 0)),
                  pl.BlockSpec((1, D), lambda g: (0, 0)),
                  _full(p['ta_in_W']),
                  pl.BlockSpec((1, 3 * D), lambda g: (0, 0)),
                  _full(p['ta_out_W']),
                  pl.BlockSpec((1, D), lambda g: (0, 0)),
                  _full(p['gate_W']),
                  pl.BlockSpec((1, D), lambda g: (0, 0))],
        out_specs=pl.BlockSpec((RT, T, D), lambda g: (g, 0, 0)),
        out_shape=jax.ShapeDtypeStruct((B * N, T, D), f32),
    )
    st = temporal_call(stin, x0r, p['skip_W'], row2(p['skip_b']),
                       row2(p['tln_w']), row2(p['tln_b']),
                       p['ta_in_W'], row2(p['ta_in_b']),
                       p['ta_out_W'], row2(p['ta_out_b']),
                       p['gate_W'], row2(p['gate_b']))

    s2 = st.reshape(B, N, T, D).transpose(0, 2, 1, 3).reshape(B * T, N, D)
    station_call = pl.pallas_call(
        _station_body,
        grid=(B * T,),
        in_specs=[pl.BlockSpec((1, N, D), lambda g: (g, 0, 0)),
                  pl.BlockSpec((1, D), lambda g: (0, 0)),
                  pl.BlockSpec((1, D), lambda g: (0, 0)),
                  _full(p['sa_in_W']),
                  pl.BlockSpec((1, 3 * D), lambda g: (0, 0)),
                  _full(p['sa_out_W']),
                  pl.BlockSpec((1, D), lambda g: (0, 0))],
        out_specs=pl.BlockSpec((1, N, D), lambda g: (g, 0, 0)),
        out_shape=jax.ShapeDtypeStruct((B * T, N, D), f32),
    )
    stat = station_call(s2, row2(p['sln_w']), row2(p['sln_b']),
                        p['sa_in_W'], row2(p['sa_in_b']),
                        p['sa_out_W'], row2(p['sa_out_b']))

    st_btnd = st.reshape(B, N, T, D).transpose(0, 2, 1, 3).reshape(B * T * N, D)
    statf = stat.reshape(B * T * N, D)
    NB = B * T * N // RC
    ro1_call = pl.pallas_call(
        _ro1_body,
        grid=(NB,),
        in_specs=[pl.BlockSpec((RC, D), lambda g: (g, 0)),
                  pl.BlockSpec((RC, D), lambda g: (g, 0)),
                  _full(p['ro_W1']), pl.BlockSpec((1, H), lambda g: (0, 0))],
        out_specs=[pl.BlockSpec((RC, H), lambda g: (g, 0)),
                   pl.BlockSpec((1, 1, H), lambda g: (g, 0, 0)),
                   pl.BlockSpec((1, 1, H), lambda g: (g, 0, 0))],
        out_shape=[jax.ShapeDtypeStruct((B * T * N, H), f32),
                   jax.ShapeDtypeStruct((NB, 1, H), f32),
                   jax.ShapeDtypeStruct((NB, 1, H), f32)],
    )
    r, sums, sumsqs = ro1_call(st_btnd, statf, p['ro_W1'], row2(p['ro_b1']))

    ro2_call = pl.pallas_call(
        _ro2_body,
        grid=(NB,),
        in_specs=[pl.BlockSpec((RC, H), lambda g: (g, 0)),
                  _full(sums), _full(sumsqs),
                  pl.BlockSpec((1, H), lambda g: (0, 0)),
                  pl.BlockSpec((1, H), lambda g: (0, 0)),
                  _full(p['ro_W2']),
                  pl.BlockSpec((1, H), lambda g: (0, 0)),
                  pl.BlockSpec((1, H), lambda g: (0, 0)),
                  pl.BlockSpec((1, 1), lambda g: (0, 0)),
                  pl.BlockSpec((1, H), lambda g: (0, 0)),
                  pl.BlockSpec((1, 1), lambda g: (0, 0))],
        out_specs=pl.BlockSpec((RC, 2), lambda g: (g, 0)),
        out_shape=jax.ShapeDtypeStruct((B * T * N, 2), f32),
    )
    outf = ro2_call(r, sums, sumsqs, row2(p['bn_w']), row2(p['bn_b']),
                    p['ro_W2'], row2(p['ro_b2']),
                    p['loc_W'].reshape(1, H), p['loc_b'].reshape(1, 1),
                    p['scale_W'].reshape(1, H), p['scale_b'].reshape(1, 1))
    return outf.reshape(B, T, N, 2)
